# baseline (speedup 1.0000x reference)
import functools

import jax
import jax.numpy as jnp
from jax import lax
from jax.experimental import pallas as pl
from jax.experimental.pallas import tpu as pltpu
from jax.experimental.pallas import tpu_sc as plsc

T = 2048
D = 1024
E = 8
F = 512
K = 2
BM = 256
NB = T * K // BM + E
NBTOT = NB * BM

NW = 32
TPW = T // NW
PPW = TPW * E
SPW = TPW * K


def _router_body(x_ref, gw_ref, xbf_ref, p2_ref, wv2_ref,
                 eb_ref, tp_ref):
    xbf_ref[...] = x_ref[...].astype(jnp.bfloat16)
    logits = lax.dot_general(
        x_ref[...], gw_ref[...], (((1,), (1,)), ((), ())),
        preferred_element_type=jnp.float32)
    m = jnp.max(logits, axis=1, keepdims=True)
    ex = jnp.exp(logits - m)
    probs = ex / jnp.sum(ex, axis=1, keepdims=True)
    lane = lax.broadcasted_iota(jnp.int32, (T, E), 1)
    rank = jnp.zeros((T, E), jnp.int32)
    for j in range(E):
        pj = probs[:, j:j + 1]
        beats = (pj > probs) | ((pj == probs) & (j < lane))
        rank = rank + beats.astype(jnp.int32)
    sel = rank < K
    selF = sel.astype(jnp.float32)
    comb = probs * selF

    rowi = lax.broadcasted_iota(jnp.int32, (T, T), 0)
    coli = lax.broadcasted_iota(jnp.int32, (T, T), 1)
    tril = (coli < rowi).astype(jnp.float32)
    rank_t = lax.dot_general(tril, selF, (((1,), (0,)), ((), ())),
                             preferred_element_type=jnp.float32)
    rank_t = rank_t.astype(jnp.int32)

    counts = jnp.sum(selF, axis=0, keepdims=True).astype(jnp.int32)
    pcnt = ((counts + BM - 1) // BM) * BM
    lane1 = lax.broadcasted_iota(jnp.int32, (1, E), 1)
    poff = jnp.zeros((1, E), jnp.int32)
    for j in range(E):
        poff = poff + jnp.where(lane1 > j, pcnt[:, j:j + 1], 0)

    pos = jnp.where(sel, poff + rank_t, NBTOT)

    big = jnp.int32(NBTOT + 1)
    p0 = jnp.min(jnp.where(sel, pos, big), axis=1, keepdims=True)
    p1 = jnp.max(jnp.where(sel, pos, -1), axis=1, keepdims=True)
    w0 = jnp.sum(jnp.where(pos == p0, comb, 0.0), axis=1, keepdims=True)
    w1v = jnp.sum(jnp.where(pos == p1, comb, 0.0), axis=1, keepdims=True)
    p2_ref[...] = jnp.concatenate([p0, p1], axis=1)
    wv2_ref[...] = jnp.concatenate(
        [jnp.broadcast_to(w0, (T, 16)), jnp.broadcast_to(w1v, (T, 16))],
        axis=1)

    bidx = lax.broadcasted_iota(jnp.int32, (1, NB), 1)
    eb = jnp.zeros((1, NB), jnp.int32)
    for j in range(E):
        endb = (poff[:, j:j + 1] + pcnt[:, j:j + 1]) // BM
        eb = eb + (bidx >= endb).astype(jnp.int32)
    eb_ref[...] = jnp.minimum(eb, E - 1)
    tp_ref[...] = jnp.sum(pcnt, axis=1, keepdims=True)


def _dispatch_body(p2f_hbm, xbf_hbm, xg_hbm, ctok, cpos, rows, sem1, sem2):
    wid = lax.axis_index("s") * 2 + lax.axis_index("c")
    tbase = wid * TPW
    pltpu.sync_copy(p2f_hbm.at[pl.ds(wid * SPW, SPW)], cpos)
    iota16 = lax.iota(jnp.int32, 16)
    for i in range(SPW // 16):
        ctok[pl.ds(16 * i, 16)] = tbase + jax.lax.shift_right_logical(
            16 * i + iota16, 1)
    pltpu.async_copy(xbf_hbm.at[ctok], rows, sem1).wait()
    pltpu.async_copy(rows, xg_hbm.at[cpos], sem2).wait()


def _make_dispatch():
    mesh = plsc.VectorSubcoreMesh(core_axis_name="c", subcore_axis_name="s")
    return functools.partial(
        pl.kernel, mesh=mesh,
        out_type=jax.ShapeDtypeStruct((NBTOT, D // 2), jnp.int32),
        scratch_types=[
            pltpu.VMEM((SPW,), jnp.int32),
            pltpu.VMEM((SPW,), jnp.int32),
            pltpu.VMEM((SPW, D // 2), jnp.int32),
            pltpu.SemaphoreType.DMA,
            pltpu.SemaphoreType.DMA,
        ],
    )(_dispatch_body)


def _gmm_body(eb_ref, tp_ref, xg_ref, w1_ref, w3_ref, w2_ref, yg_ref):
    b = pl.program_id(0)

    @pl.when(b * BM < tp_ref[0])
    def _compute():
        xs = xg_ref[...]
        w1 = w1_ref[0].astype(jnp.bfloat16)
        w3 = w3_ref[0].astype(jnp.bfloat16)
        w2 = w2_ref[0].astype(jnp.bfloat16)
        h1 = lax.dot_general(xs, w1, (((1,), (1,)), ((), ())),
                             preferred_element_type=jnp.float32)
        h3 = lax.dot_general(xs, w3, (((1,), (1,)), ((), ())),
                             preferred_element_type=jnp.float32)
        h = (h1 * jax.nn.sigmoid(h1) * h3).astype(jnp.bfloat16)
        yg_ref[...] = lax.dot_general(h, w2, (((1,), (1,)), ((), ())),
                                      preferred_element_type=jnp.float32)


def _combine_body(p2f_hbm, wbf_hbm, yg_hbm, out_hbm, pv, wvb, rows, ov, sem):
    wid = lax.axis_index("s") * 2 + lax.axis_index("c")
    tbase = wid * TPW
    pltpu.sync_copy(wbf_hbm.at[pl.ds(K * tbase * 16, K * TPW * 16)], wvb)
    htok = TPW // 2
    for half in range(2):
        pltpu.sync_copy(
            p2f_hbm.at[pl.ds(K * tbase + K * htok * half, K * htok)], pv)
        pltpu.async_copy(yg_hbm.at[pv], rows, sem).wait()

        def body(j, _):
            c = K * htok * half + 2 * j
            w0 = wvb[pl.ds(16 * c, 16)]
            w1v = wvb[pl.ds(16 * (c + 1), 16)]
            for cc in range(D // 16):
                a = rows[2 * j, pl.ds(16 * cc, 16)]
                bb = rows[2 * j + 1, pl.ds(16 * cc, 16)]
                ov[j, pl.ds(16 * cc, 16)] = a * w0 + bb * w1v
            return 0

        lax.fori_loop(0, htok, body, 0)
        pltpu.sync_copy(ov, out_hbm.at[pl.ds(tbase + htok * half, htok)])


def _make_combine():
    mesh = plsc.VectorSubcoreMesh(core_axis_name="c", subcore_axis_name="s")
    return functools.partial(
        pl.kernel, mesh=mesh,
        out_type=jax.ShapeDtypeStruct((T, D), jnp.float32),
        scratch_types=[
            pltpu.VMEM((K * TPW // 2,), jnp.int32),
            pltpu.VMEM((K * TPW * 16,), jnp.float32),
            pltpu.VMEM((K * TPW // 2, D), jnp.float32),
            pltpu.VMEM((TPW // 2, D), jnp.float32),
            pltpu.SemaphoreType.DMA,
        ],
    )(_combine_body)


def _moe(hidden_states, gate_w, w1, w3, w2):
    x = hidden_states.astype(jnp.float32)
    xbf, p2, wv2, eb, tp = pl.pallas_call(
        _router_body,
        out_shape=(
            jax.ShapeDtypeStruct((T, D), jnp.bfloat16),
            jax.ShapeDtypeStruct((T, K), jnp.int32),
            jax.ShapeDtypeStruct((T, K * 16), jnp.float32),
            jax.ShapeDtypeStruct((1, NB), jnp.int32),
            jax.ShapeDtypeStruct((1, 1), jnp.int32),
        ),
    )(x, gate_w.astype(jnp.float32))

    xbf32 = lax.bitcast_convert_type(
        xbf.reshape(T, D // 2, 2), jnp.int32)
    xg32 = _make_dispatch()(p2.reshape(-1), xbf32)
    xg = lax.bitcast_convert_type(xg32, jnp.bfloat16).reshape(NBTOT, D)

    grid_spec = pltpu.PrefetchScalarGridSpec(
        num_scalar_prefetch=2,
        grid=(NB,),
        in_specs=[
            pl.BlockSpec((BM, D), lambda b, eb, tp: (b, 0)),
            pl.BlockSpec((1, F, D), lambda b, eb, tp: (eb[b], 0, 0)),
            pl.BlockSpec((1, F, D), lambda b, eb, tp: (eb[b], 0, 0)),
            pl.BlockSpec((1, D, F), lambda b, eb, tp: (eb[b], 0, 0)),
        ],
        out_specs=pl.BlockSpec((BM, D), lambda b, eb, tp: (b, 0)),
    )
    yg = pl.pallas_call(
        _gmm_body,
        grid_spec=grid_spec,
        out_shape=jax.ShapeDtypeStruct((NBTOT, D), jnp.float32),
        compiler_params=pltpu.CompilerParams(
            dimension_semantics=("arbitrary",)),
    )(eb.reshape(-1), tp.reshape(-1), xg, w1, w3, w2)

    out = _make_combine()(p2.reshape(-1), wv2.reshape(-1), yg)
    return out


def kernel(hidden_states, num_global_tokens, max_num_tokens_per_gpu,
           gate_w, w1, w3, w2):
    del num_global_tokens, max_num_tokens_per_gpu
    return _moe(hidden_states, gate_w, w1, w3, w2)


# speedup vs baseline: 2.2653x; 2.2653x over previous
import functools

import jax
import jax.numpy as jnp
from jax import lax
from jax.experimental import pallas as pl
from jax.experimental.pallas import tpu as pltpu
from jax.experimental.pallas import tpu_sc as plsc

T = 2048
D = 1024
E = 8
F = 512
K = 2
BM = 256
NB = T * K // BM + E
NBTOT = NB * BM

NW = 32
TPW = T // NW
SPW = TPW * K
RND = 4
RPR = SPW // RND
QT = 4
QTOK = TPW // QT


def _router_body(x_ref, gw_ref, p2_ref, wv2_ref, eb_ref, tp_ref):
    logits = lax.dot_general(
        x_ref[...], gw_ref[...], (((1,), (1,)), ((), ())),
        preferred_element_type=jnp.float32)
    m = jnp.max(logits, axis=1, keepdims=True)
    ex = jnp.exp(logits - m)
    probs = ex / jnp.sum(ex, axis=1, keepdims=True)
    lane = lax.broadcasted_iota(jnp.int32, (T, E), 1)
    rank = jnp.zeros((T, E), jnp.int32)
    for j in range(E):
        pj = probs[:, j:j + 1]
        beats = (pj > probs) | ((pj == probs) & (j < lane))
        rank = rank + beats.astype(jnp.int32)
    sel = rank < K
    selF = sel.astype(jnp.float32)
    comb = probs * selF

    rowi = lax.broadcasted_iota(jnp.int32, (T, T), 0)
    coli = lax.broadcasted_iota(jnp.int32, (T, T), 1)
    tril = (coli < rowi).astype(jnp.float32)
    rank_t = lax.dot_general(tril, selF, (((1,), (0,)), ((), ())),
                             preferred_element_type=jnp.float32)
    rank_t = rank_t.astype(jnp.int32)

    counts = jnp.sum(selF, axis=0, keepdims=True).astype(jnp.int32)
    pcnt = ((counts + BM - 1) // BM) * BM
    lane1 = lax.broadcasted_iota(jnp.int32, (1, E), 1)
    poff = jnp.zeros((1, E), jnp.int32)
    for j in range(E):
        poff = poff + jnp.where(lane1 > j, pcnt[:, j:j + 1], 0)

    pos = jnp.where(sel, poff + rank_t, NBTOT)

    big = jnp.int32(NBTOT + 1)
    p0 = jnp.min(jnp.where(sel, pos, big), axis=1, keepdims=True)
    p1 = jnp.max(jnp.where(sel, pos, -1), axis=1, keepdims=True)
    w0 = jnp.sum(jnp.where(pos == p0, comb, 0.0), axis=1, keepdims=True)
    w1v = jnp.sum(jnp.where(pos == p1, comb, 0.0), axis=1, keepdims=True)
    p2_ref[...] = jnp.concatenate([p0, p1], axis=1)
    wv2_ref[...] = jnp.concatenate(
        [jnp.broadcast_to(w0, (T, 16)), jnp.broadcast_to(w1v, (T, 16))],
        axis=1)

    bidx = lax.broadcasted_iota(jnp.int32, (1, NB), 1)
    eb = jnp.zeros((1, NB), jnp.int32)
    for j in range(E):
        endb = (poff[:, j:j + 1] + pcnt[:, j:j + 1]) // BM
        eb = eb + (bidx >= endb).astype(jnp.int32)
    eb_ref[...] = jnp.minimum(eb, E - 1)
    tp_ref[...] = jnp.sum(pcnt, axis=1, keepdims=True)


def _dispatch_body(p2f_hbm, x_hbm, xg_hbm,
                   ctok0, ctok1, cpos0, cpos1, rows0, rows1,
                   gsem0, gsem1, ssem0, ssem1):
    wid = lax.axis_index("s") * 2 + lax.axis_index("c")
    tbase = wid * TPW
    iota16 = lax.iota(jnp.int32, 16)
    ctoks = (ctok0, ctok1)
    cposs = (cpos0, cpos1)
    rows = (rows0, rows1)
    gsems = (gsem0, gsem1)
    ssems = (ssem0, ssem1)

    def fill(r):
        p = r & 1
        pltpu.sync_copy(p2f_hbm.at[pl.ds(wid * SPW + r * RPR, RPR)], cposs[p])
        for i in range(RPR // 16):
            ctoks[p][pl.ds(16 * i, 16)] = (
                tbase + jax.lax.shift_right_logical(
                    r * RPR + 16 * i + iota16, 1))
        pltpu.async_copy(x_hbm.at[ctoks[p]], rows[p], gsems[p])

    fill(0)
    for r in range(RND):
        p = r & 1
        pltpu.make_async_copy(x_hbm.at[ctoks[p]], rows[p], gsems[p]).wait()
        if r + 1 < RND:
            if r >= 1:
                pltpu.make_async_copy(
                    rows[1 - p], xg_hbm.at[cposs[1 - p]], ssems[1 - p]).wait()
            fill(r + 1)
        pltpu.async_copy(rows[p], xg_hbm.at[cposs[p]], ssems[p])
    pltpu.make_async_copy(rows[0], xg_hbm.at[cposs[0]], ssems[0]).wait()
    pltpu.make_async_copy(rows[1], xg_hbm.at[cposs[1]], ssems[1]).wait()


def _make_dispatch():
    mesh = plsc.VectorSubcoreMesh(core_axis_name="c", subcore_axis_name="s")
    return functools.partial(
        pl.kernel, mesh=mesh,
        out_type=jax.ShapeDtypeStruct((NBTOT, D), jnp.float32),
        scratch_types=[
            pltpu.VMEM((RPR,), jnp.int32),
            pltpu.VMEM((RPR,), jnp.int32),
            pltpu.VMEM((RPR,), jnp.int32),
            pltpu.VMEM((RPR,), jnp.int32),
            pltpu.VMEM((RPR, D), jnp.float32),
            pltpu.VMEM((RPR, D), jnp.float32),
            pltpu.SemaphoreType.DMA,
            pltpu.SemaphoreType.DMA,
            pltpu.SemaphoreType.DMA,
            pltpu.SemaphoreType.DMA,
        ],
    )(_dispatch_body)


def _gmm_body(eb_ref, tp_ref, xg_ref, w1_ref, w3_ref, w2_ref, yg_ref):
    b = pl.program_id(0)

    @pl.when(b * BM < tp_ref[0])
    def _compute():
        xs = xg_ref[...].astype(jnp.bfloat16)
        w1 = w1_ref[0].astype(jnp.bfloat16)
        w3 = w3_ref[0].astype(jnp.bfloat16)
        w2 = w2_ref[0].astype(jnp.bfloat16)
        h1 = lax.dot_general(xs, w1, (((1,), (1,)), ((), ())),
                             preferred_element_type=jnp.float32)
        h3 = lax.dot_general(xs, w3, (((1,), (1,)), ((), ())),
                             preferred_element_type=jnp.float32)
        h = (h1 * jax.nn.sigmoid(h1) * h3).astype(jnp.bfloat16)
        yg_ref[...] = lax.dot_general(h, w2, (((1,), (1,)), ((), ())),
                                      preferred_element_type=jnp.float32)


def _combine_body(p2f_hbm, wbf_hbm, yg_hbm, out_hbm,
                  pq0, pq1, wvb, rows0, rows1, ov, gsem0, gsem1):
    wid = lax.axis_index("s") * 2 + lax.axis_index("c")
    tbase = wid * TPW
    pltpu.sync_copy(wbf_hbm.at[pl.ds(SPW * 16 * wid, SPW * 16)], wvb)
    pqs = (pq0, pq1)
    rows = (rows0, rows1)
    gsems = (gsem0, gsem1)

    def fill(q):
        p = q & 1
        pltpu.sync_copy(
            p2f_hbm.at[pl.ds(wid * SPW + q * K * QTOK, K * QTOK)], pqs[p])
        pltpu.async_copy(yg_hbm.at[pqs[p]], rows[p], gsems[p])

    fill(0)
    for q in range(QT):
        p = q & 1
        pltpu.make_async_copy(yg_hbm.at[pqs[p]], rows[p], gsems[p]).wait()
        if q + 1 < QT:
            fill(q + 1)

        def body(j, _):
            c = q * K * QTOK + 2 * j
            w0 = wvb[pl.ds(16 * c, 16)]
            w1v = wvb[pl.ds(16 * (c + 1), 16)]
            for cc in range(D // 16):
                a = rows[p][2 * j, pl.ds(16 * cc, 16)]
                bb = rows[p][2 * j + 1, pl.ds(16 * cc, 16)]
                ov[j, pl.ds(16 * cc, 16)] = a * w0 + bb * w1v
            return 0

        lax.fori_loop(0, QTOK, body, 0)
        pltpu.sync_copy(ov, out_hbm.at[pl.ds(tbase + QTOK * q, QTOK)])


def _make_combine():
    mesh = plsc.VectorSubcoreMesh(core_axis_name="c", subcore_axis_name="s")
    return functools.partial(
        pl.kernel, mesh=mesh,
        out_type=jax.ShapeDtypeStruct((T, D), jnp.float32),
        scratch_types=[
            pltpu.VMEM((K * QTOK,), jnp.int32),
            pltpu.VMEM((K * QTOK,), jnp.int32),
            pltpu.VMEM((SPW * 16,), jnp.float32),
            pltpu.VMEM((K * QTOK, D), jnp.float32),
            pltpu.VMEM((K * QTOK, D), jnp.float32),
            pltpu.VMEM((QTOK, D), jnp.float32),
            pltpu.SemaphoreType.DMA,
            pltpu.SemaphoreType.DMA,
        ],
    )(_combine_body)


def _moe(hidden_states, gate_w, w1, w3, w2):
    x = hidden_states.astype(jnp.float32)
    p2, wv2, eb, tp = pl.pallas_call(
        _router_body,
        out_shape=(
            jax.ShapeDtypeStruct((T, K), jnp.int32),
            jax.ShapeDtypeStruct((T, K * 16), jnp.float32),
            jax.ShapeDtypeStruct((1, NB), jnp.int32),
            jax.ShapeDtypeStruct((1, 1), jnp.int32),
        ),
    )(x, gate_w.astype(jnp.float32))

    p2f = p2.reshape(-1)
    xg = _make_dispatch()(p2f, x)

    grid_spec = pltpu.PrefetchScalarGridSpec(
        num_scalar_prefetch=2,
        grid=(NB,),
        in_specs=[
            pl.BlockSpec((BM, D), lambda b, eb, tp: (b, 0)),
            pl.BlockSpec((1, F, D), lambda b, eb, tp: (eb[b], 0, 0)),
            pl.BlockSpec((1, F, D), lambda b, eb, tp: (eb[b], 0, 0)),
            pl.BlockSpec((1, D, F), lambda b, eb, tp: (eb[b], 0, 0)),
        ],
        out_specs=pl.BlockSpec((BM, D), lambda b, eb, tp: (b, 0)),
    )
    yg = pl.pallas_call(
        _gmm_body,
        grid_spec=grid_spec,
        out_shape=jax.ShapeDtypeStruct((NBTOT, D), jnp.float32),
        compiler_params=pltpu.CompilerParams(
            dimension_semantics=("arbitrary",)),
    )(eb.reshape(-1), tp.reshape(-1), xg, w1, w3, w2)

    out = _make_combine()(p2f, wv2.reshape(-1), yg)
    return out


def kernel(hidden_states, num_global_tokens, max_num_tokens_per_gpu,
           gate_w, w1, w3, w2):
    del num_global_tokens, max_num_tokens_per_gpu
    return _moe(hidden_states, gate_w, w1, w3, w2)


# speedup vs baseline: 3.5868x; 1.5834x over previous
import jax
import jax.numpy as jnp
from jax import lax
from jax.experimental import pallas as pl
from jax.experimental.pallas import tpu as pltpu

T = 2048
D = 1024
E = 8
F = 512


def _moe_body(x_ref, gw_ref, w1_ref, w3_ref, w2_ref, out_ref,
              comb_ref, xbf_ref, ha_ref, hb_ref):
    e = pl.program_id(0)

    @pl.when(e == 0)
    def _router():
        xbf_ref[...] = x_ref[...].astype(jnp.bfloat16)
        logits = lax.dot_general(
            x_ref[...], gw_ref[...], (((1,), (1,)), ((), ())),
            preferred_element_type=jnp.float32)
        m = jnp.max(logits, axis=1, keepdims=True)
        ex = jnp.exp(logits - m)
        probs = ex / jnp.sum(ex, axis=1, keepdims=True)
        lane = lax.broadcasted_iota(jnp.int32, (T, E), 1)
        rank = jnp.zeros((T, E), jnp.int32)
        for j in range(E):
            pj = probs[:, j:j + 1]
            beats = (pj > probs) | ((pj == probs) & (j < lane))
            rank = rank + beats.astype(jnp.int32)
        comb_ref[...] = probs * (rank < 2).astype(jnp.float32)

    @pl.when(e > 0)
    def _combine():
        w2 = w2_ref[0].astype(jnp.bfloat16)
        lane = lax.broadcasted_iota(jnp.int32, (T, E), 1)
        w_prev = jnp.sum(
            jnp.where(lane == e - 1, comb_ref[...], 0.0), axis=1,
            keepdims=True)

        def acc_from(h_ref):
            y = lax.dot_general(h_ref[...], w2, (((1,), (1,)), ((), ())),
                                preferred_element_type=jnp.float32)
            yw = y * w_prev

            @pl.when(e == 1)
            def _init():
                out_ref[...] = yw

            @pl.when(e > 1)
            def _acc():
                out_ref[...] = out_ref[...] + yw

        @pl.when((e - 1) % 2 == 0)
        def _even():
            acc_from(ha_ref)

        @pl.when((e - 1) % 2 == 1)
        def _odd():
            acc_from(hb_ref)

    @pl.when(e < E)
    def _hidden():
        xs = xbf_ref[...]
        w1 = w1_ref[0].astype(jnp.bfloat16)
        w3 = w3_ref[0].astype(jnp.bfloat16)
        h1 = lax.dot_general(xs, w1, (((1,), (1,)), ((), ())),
                             preferred_element_type=jnp.float32)
        h3 = lax.dot_general(xs, w3, (((1,), (1,)), ((), ())),
                             preferred_element_type=jnp.float32)
        h = (h1 * jax.nn.sigmoid(h1) * h3).astype(jnp.bfloat16)

        @pl.when(e % 2 == 0)
        def _even():
            ha_ref[...] = h

        @pl.when(e % 2 == 1)
        def _odd():
            hb_ref[...] = h


def _moe(hidden_states, gate_w, w1, w3, w2):
    x = hidden_states.astype(jnp.float32)
    out = pl.pallas_call(
        _moe_body,
        grid=(E + 1,),
        in_specs=[
            pl.BlockSpec((T, D), lambda e: (0, 0)),
            pl.BlockSpec((E, D), lambda e: (0, 0)),
            pl.BlockSpec((1, F, D), lambda e: (jnp.minimum(e, E - 1), 0, 0)),
            pl.BlockSpec((1, F, D), lambda e: (jnp.minimum(e, E - 1), 0, 0)),
            pl.BlockSpec((1, D, F), lambda e: (jnp.maximum(e - 1, 0), 0, 0)),
        ],
        out_specs=pl.BlockSpec((T, D), lambda e: (0, 0)),
        out_shape=jax.ShapeDtypeStruct((T, D), jnp.float32),
        scratch_shapes=[pltpu.VMEM((T, E), jnp.float32),
                        pltpu.VMEM((T, D), jnp.bfloat16),
                        pltpu.VMEM((T, F), jnp.bfloat16),
                        pltpu.VMEM((T, F), jnp.bfloat16)],
        compiler_params=pltpu.CompilerParams(
            dimension_semantics=("arbitrary",)),
    )(x, gate_w.astype(jnp.float32), w1, w3, w2)
    return out


def kernel(hidden_states, num_global_tokens, max_num_tokens_per_gpu,
           gate_w, w1, w3, w2):
    del num_global_tokens, max_num_tokens_per_gpu
    return _moe(hidden_states, gate_w, w1, w3, w2)
